# baseline (speedup 1.0000x reference)
import functools

import jax
import jax.numpy as jnp
from jax import lax
from jax.experimental import pallas as pl
from jax.experimental.pallas import tpu as pltpu
from jax.experimental.pallas import tpu_sc as plsc

_N, _E, _D = 10000, 320000, 128
_TILES = 32
_NPAD = 10240
_RPT = _NPAD // 16
_CH = 128
_NCH2 = 2 * (-(-_E // (_TILES * _CH)) * _TILES // 16 // 2)
_NCH0 = 98
_NCH1 = _NCH2 - _NCH0
_EPAD = 16 * _NCH2 * _CH

_BN = 1024


@functools.cache
def _sc_aggregate(with_deg):
    mesh = plsc.VectorSubcoreMesh(
        core_axis_name="c", subcore_axis_name="s", num_cores=2, num_subcores=16
    )

    def body(h_hbm, src_hbm, zrow_hbm, zdeg_hbm, ones_hbm, *rest):
        if with_deg:
            (acc_out, deg_out,
             pq_v, rows_v, ones_v, acc_sh, deg_sh, sem) = rest
        else:
            (acc_out, pq_v, rows_v, ones_v, acc_sh, sem) = rest
        c = lax.axis_index("c")
        s = lax.axis_index("s")
        tid = c * 16 + s
        pltpu.sync_copy(ones_hbm, ones_v)
        pltpu.sync_copy(zrow_hbm, acc_sh.at[pl.ds(s * _RPT, _RPT)])
        if with_deg:
            pltpu.sync_copy(zdeg_hbm, deg_sh.at[pl.ds(s * _RPT, _RPT)])
        plsc.subcore_barrier()

        my_nch = jnp.where(c == 0, _NCH0, _NCH1)

        cbase = jnp.where(c == 0, s * _NCH0, 16 * _NCH0 + s * _NCH1)

        def chunk(i, carry):
            pltpu.sync_copy(src_hbm.at[cbase + i], pq_v)
            pltpu.async_copy(h_hbm.at[pq_v.at[0]], rows_v, sem).wait()
            pltpu.sync_copy(rows_v, acc_sh.at[pq_v.at[1]], add=True)
            if with_deg:
                pltpu.sync_copy(ones_v, deg_sh.at[pq_v.at[1]], add=True)
            return carry

        lax.fori_loop(0, my_nch, chunk, 0)
        plsc.subcore_barrier()
        pltpu.sync_copy(acc_sh.at[pl.ds(s * _RPT, _RPT)],
                        acc_out.at[c, pl.ds(s * _RPT, _RPT)])
        if with_deg:
            pltpu.sync_copy(deg_sh.at[pl.ds(s * _RPT, _RPT)],
                            deg_out.at[c, pl.ds(s * _RPT, _RPT)])

    out_type = [jax.ShapeDtypeStruct((2, _NPAD, _D), jnp.float32)]
    scratch = [
        pltpu.VMEM((2, _CH), jnp.int32),
        pltpu.VMEM((_CH, _D), jnp.float32),
        pltpu.VMEM((_CH,), jnp.float32),
        pltpu.VMEM_SHARED((_NPAD, _D), jnp.float32),
    ]
    if with_deg:
        out_type.append(jax.ShapeDtypeStruct((2, _NPAD), jnp.float32))
        scratch.append(pltpu.VMEM_SHARED((_NPAD,), jnp.float32))
    scratch.append(pltpu.SemaphoreType.DMA)
    return pl.kernel(body, out_type=tuple(out_type), mesh=mesh,
                     scratch_types=scratch)


def _layer_body(relu, h_ref, acc_ref, d0_ref, d1_ref, ws_ref, wn_ref, b_ref,
                o_ref):
    inv = 1.0 / jnp.maximum(d0_ref[...] + d1_ref[...], 1.0)
    hn = (acc_ref[0] + acc_ref[1]) * inv
    out = (jnp.dot(h_ref[...], ws_ref[...], preferred_element_type=jnp.float32)
           + jnp.dot(hn, wn_ref[...], preferred_element_type=jnp.float32)
           + b_ref[...])
    if relu:
        out = jnp.maximum(out, 0.0)
    o_ref[...] = out


def _layer_call(h, acc, d0, d1, ws, wn, b, relu):
    return pl.pallas_call(
        functools.partial(_layer_body, relu),
        grid=(_NPAD // _BN,),
        in_specs=[
            pl.BlockSpec((_BN, _D), lambda i: (i, 0)),
            pl.BlockSpec((2, _BN, _D), lambda i: (0, i, 0)),
            pl.BlockSpec((_BN, 1), lambda i: (i, 0)),
            pl.BlockSpec((_BN, 1), lambda i: (i, 0)),
            pl.BlockSpec((_D, _D), lambda i: (0, 0)),
            pl.BlockSpec((_D, _D), lambda i: (0, 0)),
            pl.BlockSpec((1, _D), lambda i: (0, 0)),
        ],
        out_specs=pl.BlockSpec((_BN, _D), lambda i: (i, 0)),
        out_shape=jax.ShapeDtypeStruct((_NPAD, _D), jnp.float32),
    )(h, acc, d0, d1, ws, wn, b)


def kernel(x, edge_index, W_self1, W_neigh1, b1, W_self2, W_neigh2, b2):
    src = edge_index[0]
    dst = edge_index[1]
    pad_e = _EPAD - _E
    src_p = jnp.concatenate([src, jnp.zeros((pad_e,), jnp.int32)])
    dst_p = jnp.concatenate([dst, jnp.full((pad_e,), _N, jnp.int32)])
    pairs = jnp.stack([src_p.reshape(-1, _CH), dst_p.reshape(-1, _CH)],
                      axis=1)
    x_p = jnp.pad(x, ((0, _NPAD - _N), (0, 0)))
    zrow = jnp.zeros((_RPT, _D), jnp.float32)
    zdeg = jnp.zeros((_RPT,), jnp.float32)
    ones = jnp.ones((_CH,), jnp.float32)

    acc1, deg = _sc_aggregate(True)(x_p, pairs, zrow, zdeg, ones)
    d0 = deg[0].reshape(_NPAD, 1)
    d1 = deg[1].reshape(_NPAD, 1)
    h1 = _layer_call(x_p, acc1, d0, d1, W_self1, W_neigh1,
                     b1.reshape(1, _D), relu=True)
    (acc2,) = _sc_aggregate(False)(h1, pairs, zrow, zdeg, ones)
    h2 = _layer_call(h1, acc2, d0, d1, W_self2, W_neigh2,
                     b2.reshape(1, _D), relu=False)

    fl = (_N * (4 * _D * _D) + _E * 2 * _D) / 1e12
    total_flops = jnp.asarray(fl + fl, dtype=jnp.float32)
    return h2[:_N], total_flops


# speedup vs baseline: 1.1565x; 1.1565x over previous
import functools

import jax
import jax.numpy as jnp
from jax import lax
from jax.experimental import pallas as pl
from jax.experimental.pallas import tpu as pltpu
from jax.experimental.pallas import tpu_sc as plsc

_N, _E, _D = 10000, 320000, 128
_TILES = 32
_NPAD = 10240
_RPT = _NPAD // 16
_CH = 128
_NCH2 = 2 * (-(-_E // (_TILES * _CH)) * _TILES // 16 // 2)
_NCH0 = 98
_NCH1 = _NCH2 - _NCH0
_EPAD = 16 * _NCH2 * _CH

_BN = 1024


@functools.cache
def _sc_aggregate(with_deg):
    mesh = plsc.VectorSubcoreMesh(
        core_axis_name="c", subcore_axis_name="s", num_cores=2, num_subcores=16
    )

    def body(h_hbm, src_hbm, dst_hbm, zrow_hbm, zdeg_hbm, ones_hbm, *rest):
        if with_deg:
            (acc_out, deg_out, src_a, dst_a, src_b, dst_b, rows_a, rows_b,
             ones_v, acc_sh, deg_sh, sema, semb) = rest
        else:
            (acc_out, src_a, dst_a, src_b, dst_b, rows_a, rows_b,
             ones_v, acc_sh, sema, semb) = rest
        c = lax.axis_index("c")
        s = lax.axis_index("s")
        tid = c * 16 + s
        pltpu.sync_copy(ones_hbm, ones_v)
        pltpu.sync_copy(zrow_hbm, acc_sh.at[pl.ds(s * _RPT, _RPT)])
        if with_deg:
            pltpu.sync_copy(zdeg_hbm, deg_sh.at[pl.ds(s * _RPT, _RPT)])
        plsc.subcore_barrier()

        my_nch = jnp.where(c == 0, _NCH0, _NCH1)

        base = jnp.where(c == 0, s * _NCH0, 16 * _NCH0 + s * _NCH1) * _CH

        def pair(i, carry):
            offa = base + (2 * i) * _CH
            offb = base + (2 * i + 1) * _CH
            pltpu.sync_copy(src_hbm.at[pl.ds(offa, _CH)], src_a)
            pltpu.sync_copy(dst_hbm.at[pl.ds(offa, _CH)], dst_a)
            ga = pltpu.async_copy(h_hbm.at[src_a], rows_a, sema)
            pltpu.sync_copy(src_hbm.at[pl.ds(offb, _CH)], src_b)
            pltpu.sync_copy(dst_hbm.at[pl.ds(offb, _CH)], dst_b)
            gb = pltpu.async_copy(h_hbm.at[src_b], rows_b, semb)
            ga.wait()
            pltpu.sync_copy(rows_a, acc_sh.at[dst_a], add=True)
            if with_deg:
                pltpu.sync_copy(ones_v, deg_sh.at[dst_a], add=True)
            gb.wait()
            pltpu.sync_copy(rows_b, acc_sh.at[dst_b], add=True)
            if with_deg:
                pltpu.sync_copy(ones_v, deg_sh.at[dst_b], add=True)
            return carry

        lax.fori_loop(0, my_nch // 2, pair, 0)
        plsc.subcore_barrier()
        pltpu.sync_copy(acc_sh.at[pl.ds(s * _RPT, _RPT)],
                        acc_out.at[c, pl.ds(s * _RPT, _RPT)])
        if with_deg:
            pltpu.sync_copy(deg_sh.at[pl.ds(s * _RPT, _RPT)],
                            deg_out.at[c, pl.ds(s * _RPT, _RPT)])

    out_type = [jax.ShapeDtypeStruct((2, _NPAD, _D), jnp.float32)]
    scratch = [
        pltpu.VMEM((_CH,), jnp.int32),
        pltpu.VMEM((_CH,), jnp.int32),
        pltpu.VMEM((_CH,), jnp.int32),
        pltpu.VMEM((_CH,), jnp.int32),
        pltpu.VMEM((_CH, _D), jnp.float32),
        pltpu.VMEM((_CH, _D), jnp.float32),
        pltpu.VMEM((_CH,), jnp.float32),
        pltpu.VMEM_SHARED((_NPAD, _D), jnp.float32),
    ]
    if with_deg:
        out_type.append(jax.ShapeDtypeStruct((2, _NPAD), jnp.float32))
        scratch.append(pltpu.VMEM_SHARED((_NPAD,), jnp.float32))
    scratch.append(pltpu.SemaphoreType.DMA)
    scratch.append(pltpu.SemaphoreType.DMA)
    return pl.kernel(body, out_type=tuple(out_type), mesh=mesh,
                     scratch_types=scratch)


def _layer_body(relu, h_ref, acc_ref, d0_ref, d1_ref, ws_ref, wn_ref, b_ref,
                o_ref):
    inv = 1.0 / jnp.maximum(d0_ref[...] + d1_ref[...], 1.0)
    hn = (acc_ref[0] + acc_ref[1]) * inv
    out = (jnp.dot(h_ref[...], ws_ref[...], preferred_element_type=jnp.float32)
           + jnp.dot(hn, wn_ref[...], preferred_element_type=jnp.float32)
           + b_ref[...])
    if relu:
        out = jnp.maximum(out, 0.0)
    o_ref[...] = out


def _layer_call(h, acc, d0, d1, ws, wn, b, relu):
    return pl.pallas_call(
        functools.partial(_layer_body, relu),
        grid=(_NPAD // _BN,),
        in_specs=[
            pl.BlockSpec((_BN, _D), lambda i: (i, 0)),
            pl.BlockSpec((2, _BN, _D), lambda i: (0, i, 0)),
            pl.BlockSpec((_BN, 1), lambda i: (i, 0)),
            pl.BlockSpec((_BN, 1), lambda i: (i, 0)),
            pl.BlockSpec((_D, _D), lambda i: (0, 0)),
            pl.BlockSpec((_D, _D), lambda i: (0, 0)),
            pl.BlockSpec((1, _D), lambda i: (0, 0)),
        ],
        out_specs=pl.BlockSpec((_BN, _D), lambda i: (i, 0)),
        out_shape=jax.ShapeDtypeStruct((_NPAD, _D), jnp.float32),
    )(h, acc, d0, d1, ws, wn, b)


def kernel(x, edge_index, W_self1, W_neigh1, b1, W_self2, W_neigh2, b2):
    src = edge_index[0]
    dst = edge_index[1]
    pad_e = _EPAD - _E
    src_p = jnp.concatenate([src, jnp.zeros((pad_e,), jnp.int32)])
    dst_p = jnp.concatenate([dst, jnp.full((pad_e,), _N, jnp.int32)])

    x_p = jnp.pad(x, ((0, _NPAD - _N), (0, 0)))
    zrow = jnp.zeros((_RPT, _D), jnp.float32)
    zdeg = jnp.zeros((_RPT,), jnp.float32)
    ones = jnp.ones((_CH,), jnp.float32)

    acc1, deg = _sc_aggregate(True)(x_p, src_p, dst_p, zrow, zdeg, ones)
    d0 = deg[0].reshape(_NPAD, 1)
    d1 = deg[1].reshape(_NPAD, 1)
    h1 = _layer_call(x_p, acc1, d0, d1, W_self1, W_neigh1,
                     b1.reshape(1, _D), relu=True)
    (acc2,) = _sc_aggregate(False)(h1, src_p, dst_p, zrow, zdeg, ones)
    h2 = _layer_call(h1, acc2, d0, d1, W_self2, W_neigh2,
                     b2.reshape(1, _D), relu=False)

    fl = (_N * (4 * _D * _D) + _E * 2 * _D) / 1e12
    total_flops = jnp.asarray(fl + fl, dtype=jnp.float32)
    return h2[:_N], total_flops
